# Initial kernel scaffold; baseline (speedup 1.0000x reference)
import jax
import jax.numpy as jnp
from jax.experimental import pallas as pl


def kernel(q_sub, q_rel, hidden, edges, nodes, id_layer, n_layer, old_nodes_new_idx, rela_embed, Ws_attn, Wr_attn, Wqr_attn_W, Wqr_attn_b, w_alpha_W, w_alpha_b, W_h):
    raise NotImplementedError("write your pallas kernel here")


# speedup vs baseline: 2.3254x; 2.3254x over previous
import functools

import jax
import jax.numpy as jnp
from jax import lax
from jax.experimental import pallas as pl
from jax.experimental.pallas import tpu as pltpu
from jax.experimental.pallas import tpu_sc as plsc

_NC = 2
_NS = 16
_L = 16
_C = 40


def _tables_body(hidden_ref, rela_ref, ws_ref, wr_ref, wqr_ref, bqr_ref,
                 ga_ref, gr_ref, ct_ref):
    h = hidden_ref[...]
    d = h.shape[1]
    ga_ref[:, :d] = h
    ga_ref[:, d:] = jnp.dot(h, ws_ref[...], preferred_element_type=jnp.float32)
    r = rela_ref[...]
    gr_ref[:, :d] = r
    gr_ref[:, d:] = jnp.dot(r, wr_ref[...], preferred_element_type=jnp.float32)
    ct_ref[...] = (jnp.dot(r, wqr_ref[...], preferred_element_type=jnp.float32)
                   + bqr_ref[...])


def _final_body(part_ref, wh_ref, out_ref):
    agg = part_ref[0] + part_ref[1]
    out_ref[...] = jnp.dot(agg, wh_ref[...], preferred_element_type=jnp.float32)


def _sc_body(n_node, n_chunks, ga, gr, ct, qrel, idx4, wv, binit,
             alpha_out, part_out,
             wv_v, binit_v, idx4_v, qidx_v,
             ga_v, gr_v, ct_v, alpha_v, acc,
             sem0, sem1, sem2):
    cid = lax.axis_index("c")
    sid = lax.axis_index("s")
    wid = sid * _NC + cid
    ebase = wid * n_chunks

    pltpu.sync_copy(wv, wv_v)
    pltpu.sync_copy(binit, binit_v)

    rbig = 640
    rlast = n_node - (_NS - 1) * rbig
    rbase = sid * rbig

    def _zrow(i, _):
        ct_v[i // 8, pl.ds((i % 8) * _L, _L)] = jnp.zeros((_L,), jnp.float32)
        return 0
    lax.fori_loop(0, _C * 8, _zrow, 0)

    def _zacc(k, _):
        pltpu.sync_copy(ct_v.at[pl.ds(0, _C)],
                        acc.at[pl.ds(rbase + k * _C, _C)])
        return 0

    @pl.when(sid < _NS - 1)
    def _():
        lax.fori_loop(0, rbig // _C, _zacc, 0)

    @pl.when(sid == _NS - 1)
    def _():
        lax.fori_loop(0, rlast // _C, _zacc, 0)
    plsc.subcore_barrier()

    wregs = [wv_v[pl.ds(j * _L, _L)] for j in range(8)]
    b0 = binit_v[...]
    lane0 = lax.iota(jnp.int32, _L) == 0

    def _chunk(i, _):
        chunk_id = ebase + i
        base = chunk_id * _C
        pltpu.sync_copy(idx4.at[chunk_id], idx4_v)
        cpq = pltpu.async_copy(qrel.at[idx4_v.at[2]], qidx_v, sem2)
        cp0 = pltpu.async_copy(ga.at[idx4_v.at[0]], ga_v, sem0)
        cp1 = pltpu.async_copy(gr.at[idx4_v.at[1]], gr_v, sem1)
        cpq.wait()
        cp2 = pltpu.async_copy(ct.at[qidx_v], ct_v, sem2)
        cp0.wait()
        cp1.wait()
        cp2.wait()

        def _edge(e, _c):
            accv = b0
            for j in range(8):
                a = ga_v[e, pl.ds(128 + j * _L, _L)]
                g = gr_v[e, pl.ds(128 + j * _L, _L)]
                q = ct_v[e, pl.ds(j * _L, _L)]
                t = jnp.maximum(a + g + q, 0.0)
                accv = accv + t * wregs[j]
            s = jnp.sum(accv)
            av = jnp.broadcast_to(s, (_L,))
            av = 1.0 / (1.0 + jnp.exp(-av))
            plsc.store_scatter(alpha_v, [jnp.broadcast_to(e, (_L,))], av,
                               mask=lane0)
            for j in range(8):
                h = ga_v[e, pl.ds(j * _L, _L)]
                r = gr_v[e, pl.ds(j * _L, _L)]
                ct_v[e, pl.ds(j * _L, _L)] = av * (h + r)
            return 0
        lax.fori_loop(0, _C, _edge, 0)

        pltpu.sync_copy(ct_v, acc.at[idx4_v.at[3]], add=True)
        pltpu.sync_copy(alpha_v, alpha_out.at[pl.ds(base, _C)])
        return 0
    lax.fori_loop(0, n_chunks, _chunk, 0)

    plsc.subcore_barrier()

    @pl.when(sid < _NS - 1)
    def _():
        pltpu.sync_copy(acc.at[pl.ds(rbase, rbig)],
                        part_out.at[cid, pl.ds(rbase, rbig)])

    @pl.when(sid == _NS - 1)
    def _():
        pltpu.sync_copy(acc.at[pl.ds(rbase, rlast)],
                        part_out.at[cid, pl.ds(rbase, rlast)])


def kernel(q_sub, q_rel, hidden, edges, nodes, id_layer, n_layer,
           old_nodes_new_idx, rela_embed, Ws_attn, Wr_attn, Wqr_attn_W,
           Wqr_attn_b, w_alpha_W, w_alpha_b, W_h):
    n_node, in_dim = hidden.shape
    vocab = rela_embed.shape[0]
    n_edge = edges.shape[0]
    n_workers = _NC * _NS
    assert n_edge % (n_workers * _C) == 0
    n_chunks = n_edge // (n_workers * _C)

    vp = -(-vocab // 8) * 8
    rela_p = jnp.concatenate(
        [rela_embed, jnp.zeros((vp - vocab, in_dim), jnp.float32)], axis=0)

    ga, gr, ct = pl.pallas_call(
        _tables_body,
        out_shape=[
            jax.ShapeDtypeStruct((n_node, 2 * in_dim), jnp.float32),
            jax.ShapeDtypeStruct((vp, 2 * in_dim), jnp.float32),
            jax.ShapeDtypeStruct((vp, in_dim), jnp.float32),
        ],
    )(hidden, rela_p, Ws_attn, Wr_attn, Wqr_attn_W, Wqr_attn_b.reshape(1, -1))

    idx4 = jnp.stack([edges[:, 4], edges[:, 2], edges[:, 0], edges[:, 5]])
    idx4 = (idx4.astype(jnp.int32)
            .reshape(4, n_edge // _C, _C).transpose(1, 0, 2))
    wv = w_alpha_W[:, 0]
    binit = jnp.zeros((_L,), jnp.float32).at[0].set(w_alpha_b[0])

    mesh = plsc.VectorSubcoreMesh(core_axis_name="c", subcore_axis_name="s",
                                  num_cores=_NC, num_subcores=_NS)
    sc = pl.kernel(
        functools.partial(_sc_body, n_node, n_chunks),
        out_type=[
            jax.ShapeDtypeStruct((n_edge,), jnp.float32),
            jax.ShapeDtypeStruct((_NC, n_node, in_dim), jnp.float32),
        ],
        mesh=mesh,
        compiler_params=pltpu.CompilerParams(needs_layout_passes=False),
        scratch_types=[
            pltpu.VMEM((in_dim,), jnp.float32),
            pltpu.VMEM((_L,), jnp.float32),
            pltpu.VMEM((4, _C), jnp.int32),
            pltpu.VMEM((_C,), jnp.int32),
            pltpu.VMEM((_C, 2 * in_dim), jnp.float32),
            pltpu.VMEM((_C, 2 * in_dim), jnp.float32),
            pltpu.VMEM((_C, in_dim), jnp.float32),
            pltpu.VMEM((_C,), jnp.float32),
            pltpu.VMEM_SHARED((n_node, in_dim), jnp.float32),
            pltpu.SemaphoreType.DMA,
            pltpu.SemaphoreType.DMA,
            pltpu.SemaphoreType.DMA,
        ],
    )
    alpha_flat, part = sc(ga, gr, ct, q_rel.astype(jnp.int32), idx4, wv, binit)

    hidden_new = pl.pallas_call(
        _final_body,
        out_shape=jax.ShapeDtypeStruct((n_node, W_h.shape[1]), jnp.float32),
    )(part, W_h)

    alpha = alpha_flat.reshape(n_edge, 1)
    sampled_nodes_idx = (nodes[:, 1] > -1) & (nodes[:, 1] < n_node + 1)
    final_nodes = jnp.array([0], dtype=nodes.dtype)
    return (hidden_new, nodes, final_nodes, old_nodes_new_idx,
            sampled_nodes_idx, alpha, edges)
